# Initial kernel scaffold; baseline (speedup 1.0000x reference)
import functools

import jax
import jax.numpy as jnp
from jax import lax
from jax.experimental import pallas as pl
from jax.experimental.pallas import tpu as pltpu
from jax.experimental.pallas import tpu_sc as plsc

VOCAB = 1000000
EMBED = 64

_NC = 2
_NS = 16
_NW = _NC * _NS

_C = 128


def _gather_kernel_body(b_per_w, steps, idx_hbm, table_hbm, out_hbm,
                        idx_v, rows_v, sem):
    wid = lax.axis_index("s") * _NC + lax.axis_index("c")
    base = wid * b_per_w

    def step(i, carry):
        off = base + i * _C
        pltpu.sync_copy(idx_hbm.at[pl.ds(off, _C)], idx_v)
        pltpu.async_copy(table_hbm.at[idx_v], rows_v, sem).wait()
        pltpu.sync_copy(rows_v, out_hbm.at[pl.ds(off, _C)])
        return carry

    lax.fori_loop(0, steps, step, 0)


def kernel(idx, table):
    B = idx.shape[0] * idx.shape[1]
    idx_flat = idx.reshape((B,)).astype(jnp.int32)
    b_per_w = B // _NW
    steps = b_per_w // _C

    mesh = plsc.VectorSubcoreMesh(core_axis_name="c", subcore_axis_name="s")
    k = functools.partial(
        pl.kernel,
        mesh=mesh,
        out_type=jax.ShapeDtypeStruct((B, EMBED), jnp.float32),
        scratch_types=[
            pltpu.VMEM((_C,), jnp.int32),
            pltpu.VMEM((_C, EMBED), jnp.float32),
            pltpu.SemaphoreType.DMA,
        ],
    )(functools.partial(_gather_kernel_body, b_per_w, steps))

    out_flat = k(idx_flat, table)
    return out_flat.reshape((idx.shape[0], idx.shape[1], EMBED))


# speedup vs baseline: 1.5736x; 1.5736x over previous
import functools

import jax
import jax.numpy as jnp
from jax import lax
from jax.experimental import pallas as pl
from jax.experimental.pallas import tpu as pltpu
from jax.experimental.pallas import tpu_sc as plsc

VOCAB = 1000000
EMBED = 64

_NC = 2
_NS = 16
_NW = _NC * _NS

_C = 128


def _gather_kernel_body(b_per_w, steps, idx_hbm, table_hbm, out_hbm,
                        idx_v, rows_v, sem):
    wid = lax.axis_index("s") * _NC + lax.axis_index("c")
    base = wid * b_per_w

    def step(i, carry):
        off = base + i * _C
        pltpu.sync_copy(idx_hbm.at[pl.ds(off, _C)], idx_v)
        pltpu.async_copy(table_hbm.at[idx_v], rows_v, sem).wait()
        pltpu.sync_copy(rows_v, out_hbm.at[pl.ds(off, _C)])
        return carry

    lax.fori_loop(0, steps, step, 0)


def kernel(idx, table):
    B = idx.shape[0] * idx.shape[1]
    idx_flat = idx.reshape((B,)).astype(jnp.int32)
    b_per_w = B // _NW
    steps = b_per_w // _C

    mesh = plsc.VectorSubcoreMesh(core_axis_name="c", subcore_axis_name="s")
    k = functools.partial(
        pl.kernel,
        mesh=mesh,
        out_type=jax.ShapeDtypeStruct((B, EMBED), jnp.float32),
        scratch_types=[
            pltpu.VMEM((_C,), jnp.int32),
            pltpu.VMEM((_C, EMBED), jnp.float32),
            pltpu.SemaphoreType.DMA,
        ],
        compiler_params=pltpu.CompilerParams(use_tc_tiling_on_sc=False),
    )(functools.partial(_gather_kernel_body, b_per_w, steps))

    out_flat = k(idx_flat, table)
    return out_flat.reshape((idx.shape[0], idx.shape[1], EMBED))


# speedup vs baseline: 1.8731x; 1.1903x over previous
import functools

import jax
import jax.numpy as jnp
from jax import lax
from jax.experimental import pallas as pl
from jax.experimental.pallas import tpu as pltpu
from jax.experimental.pallas import tpu_sc as plsc

VOCAB = 1000000
EMBED = 64

_NC = 2
_NS = 16
_NW = _NC * _NS

_C = 128
_K = 4
_GROUP = _K * _C


def _gather_kernel_body(b_per_w, groups, idx_hbm, table_hbm, out_hbm,
                        idx_all, rows_v, gsem0, gsem1, osem0, osem1):
    wid = lax.axis_index("s") * _NC + lax.axis_index("c")
    base = wid * b_per_w
    gsem = (gsem0, gsem1)
    osem = (osem0, osem1)

    pltpu.sync_copy(idx_hbm.at[pl.ds(base, b_per_w)], idx_all)

    def fire_gathers(g, b):
        for j in range(_K):
            pltpu.async_copy(
                table_hbm.at[idx_all.at[pl.ds(g * _GROUP + j * _C, _C)]],
                rows_v.at[b, j], gsem[b])

    def drain_gathers(b):
        for j in range(_K):
            pltpu.make_async_copy(
                table_hbm.at[pl.ds(0, _C)], rows_v.at[b, j], gsem[b]).wait()

    def fire_out(g, b):
        off = base + g * _GROUP
        for j in range(_K):
            pltpu.async_copy(
                rows_v.at[b, j], out_hbm.at[pl.ds(off + j * _C, _C)], osem[b])

    def drain_out(b):
        for j in range(_K):
            pltpu.make_async_copy(
                rows_v.at[b, j], out_hbm.at[pl.ds(0, _C)], osem[b]).wait()

    fire_gathers(0, 0)
    fire_gathers(1, 1)
    drain_gathers(0)
    fire_out(0, 0)

    def steady(m, carry):
        for t in range(2):
            g = 1 + 2 * m + t
            b = (1 + t) % 2
            nb = 1 - b
            drain_out(nb)
            fire_gathers(g + 1, nb)
            drain_gathers(b)
            fire_out(g, b)
        return carry

    lax.fori_loop(0, (groups - 2) // 2, steady, 0)

    bl = (groups - 1) % 2
    drain_out(1 - bl)
    drain_gathers(bl)
    fire_out(groups - 1, bl)
    drain_out(bl)


def kernel(idx, table):
    B = idx.shape[0] * idx.shape[1]
    idx_flat = idx.reshape((B,)).astype(jnp.int32)
    b_per_w = B // _NW
    groups = b_per_w // _GROUP

    mesh = plsc.VectorSubcoreMesh(core_axis_name="c", subcore_axis_name="s")
    k = functools.partial(
        pl.kernel,
        mesh=mesh,
        out_type=jax.ShapeDtypeStruct((B, EMBED), jnp.float32),
        scratch_types=[
            pltpu.VMEM((b_per_w,), jnp.int32),
            pltpu.VMEM((2, _K, _C, EMBED), jnp.float32),
            pltpu.SemaphoreType.DMA,
            pltpu.SemaphoreType.DMA,
            pltpu.SemaphoreType.DMA,
            pltpu.SemaphoreType.DMA,
        ],
        compiler_params=pltpu.CompilerParams(use_tc_tiling_on_sc=False),
    )(functools.partial(_gather_kernel_body, b_per_w, groups))

    out_flat = k(idx_flat, table)
    return out_flat.reshape((idx.shape[0], idx.shape[1], EMBED))


# speedup vs baseline: 1.8744x; 1.0007x over previous
import functools

import jax
import jax.numpy as jnp
from jax import lax
from jax.experimental import pallas as pl
from jax.experimental.pallas import tpu as pltpu
from jax.experimental.pallas import tpu_sc as plsc

VOCAB = 1000000
EMBED = 64

_NC = 2
_NS = 16
_NW = _NC * _NS

_C = 512
_K = 1
_GROUP = _K * _C


def _gather_kernel_body(b_per_w, groups, idx_hbm, table_hbm, out_hbm,
                        idx_all, rows_v, gsem0, gsem1, osem0, osem1):
    wid = lax.axis_index("s") * _NC + lax.axis_index("c")
    base = wid * b_per_w
    gsem = (gsem0, gsem1)
    osem = (osem0, osem1)

    pltpu.sync_copy(idx_hbm.at[pl.ds(base, b_per_w)], idx_all)

    def fire_gathers(g, b):
        for j in range(_K):
            pltpu.async_copy(
                table_hbm.at[idx_all.at[pl.ds(g * _GROUP + j * _C, _C)]],
                rows_v.at[b, j], gsem[b])

    def drain_gathers(b):
        for j in range(_K):
            pltpu.make_async_copy(
                table_hbm.at[pl.ds(0, _C)], rows_v.at[b, j], gsem[b]).wait()

    def fire_out(g, b):
        off = base + g * _GROUP
        for j in range(_K):
            pltpu.async_copy(
                rows_v.at[b, j], out_hbm.at[pl.ds(off + j * _C, _C)], osem[b])

    def drain_out(b):
        for j in range(_K):
            pltpu.make_async_copy(
                rows_v.at[b, j], out_hbm.at[pl.ds(0, _C)], osem[b]).wait()

    fire_gathers(0, 0)
    fire_gathers(1, 1)
    drain_gathers(0)
    fire_out(0, 0)

    def steady(m, carry):
        for t in range(2):
            g = 1 + 2 * m + t
            b = (1 + t) % 2
            nb = 1 - b
            drain_out(nb)
            fire_gathers(g + 1, nb)
            drain_gathers(b)
            fire_out(g, b)
        return carry

    lax.fori_loop(0, (groups - 2) // 2, steady, 0)

    bl = (groups - 1) % 2
    drain_out(1 - bl)
    drain_gathers(bl)
    fire_out(groups - 1, bl)
    drain_out(bl)


def kernel(idx, table):
    B = idx.shape[0] * idx.shape[1]
    idx_flat = idx.reshape((B,)).astype(jnp.int32)
    b_per_w = B // _NW
    groups = b_per_w // _GROUP

    mesh = plsc.VectorSubcoreMesh(core_axis_name="c", subcore_axis_name="s")
    k = functools.partial(
        pl.kernel,
        mesh=mesh,
        out_type=jax.ShapeDtypeStruct((B, EMBED), jnp.float32),
        scratch_types=[
            pltpu.VMEM((b_per_w,), jnp.int32),
            pltpu.VMEM((2, _K, _C, EMBED), jnp.float32),
            pltpu.SemaphoreType.DMA,
            pltpu.SemaphoreType.DMA,
            pltpu.SemaphoreType.DMA,
            pltpu.SemaphoreType.DMA,
        ],
        compiler_params=pltpu.CompilerParams(use_tc_tiling_on_sc=False),
    )(functools.partial(_gather_kernel_body, b_per_w, groups))

    out_flat = k(idx_flat, table)
    return out_flat.reshape((idx.shape[0], idx.shape[1], EMBED))
